# baseline (speedup 1.0000x reference)
import jax
import jax.numpy as jnp
from jax import lax
from jax.experimental import pallas as pl
from jax.experimental.pallas import tpu as pltpu

L = 1600
NCH = 128
NB = 32
F = 8
M = L // F
RP = 4
R = M + 2 * RP
FC = F * NCH
FB = F * NB
NQ = 5
KTAP = 4 * FB
NTAP = 23
BB = 8
RS = BB * R
XP = 8


def _kernel(x_ref, w1_ref, wr1_ref, wb_ref, wtap_ref, wm_ref, r2_ref, wf_ref,
            vf_ref, out_ref, xact, zbuf, zcat, xcat1):
    zbuf[0:2, :] = jnp.zeros((2, FB), jnp.bfloat16)
    zbuf[RS + 2:RS + 4, :] = jnp.zeros((2, FB), jnp.bfloat16)
    zcat[:, 96:160] = jnp.zeros((RS, 64), jnp.bfloat16)
    xact[0:XP, :] = jnp.zeros((XP, FC), jnp.bfloat16)
    xact[XP + RS:RS + 2 * XP, :] = jnp.zeros((XP, FC), jnp.bfloat16)
    xcat1[:, NQ * F + F:NCH] = jnp.zeros((RS, NCH - NQ * F - F), jnp.float32)

    for b in range(BB):
        for qi in range(NQ):
            xcat1[b * R:(b + 1) * R, F * qi:F * (qi + 1)] = \
                x_ref[b, qi:qi + R, :]
    xv = xcat1[:, 16:24]
    lf = jnp.concatenate([xcat1[:, 15:16], xcat1[:, 16:23]], axis=1)
    rf = jnp.concatenate([xcat1[:, 17:24], xcat1[:, 24:25]], axis=1)
    ki = lax.broadcasted_iota(jnp.int32, (RS, F), 0)
    ki = ki - R * (ki // R)
    ti = 8 * (ki - RP) + lax.broadcasted_iota(jnp.int32, (RS, F), 1)
    lf = jnp.where(ti == 0, -1e30, lf)
    rf = jnp.where(ti == L - 1, -1e30, rf)
    xcat1[:, NQ * F:NQ * F + F] = jnp.maximum(jnp.maximum(lf, xv), rf)
    acc = jnp.dot(xcat1[...], w1_ref[...], preferred_element_type=jnp.float32)
    z = jnp.maximum(acc + vf_ref[1:2, :], 0.0)
    res1 = jnp.dot(xv, wr1_ref[...], preferred_element_type=jnp.float32)
    res1 += vf_ref[4:5, :]

    def store_act(zval):
        xact[XP:XP + RS, :] = zval.astype(jnp.bfloat16)
        for b in range(BB):
            xact[XP + b * R:XP + b * R + RP, :] = \
                jnp.zeros((RP, FC), jnp.bfloat16)
            xact[XP + b * R + RP + M:XP + (b + 1) * R, :] = \
                jnp.zeros((RP, FC), jnp.bfloat16)

    def big(j):
        a = xact[XP:XP + RS, :]
        for s in range(F):
            zp = jnp.dot(a[:, NCH * s:NCH * (s + 1)], wb_ref[j],
                         preferred_element_type=jnp.float32)
            zbuf[2:2 + RS, NB * s:NB * (s + 1)] = zp.astype(jnp.bfloat16)
        zcat[:, 0:96] = zbuf[4:4 + RS, 0:96]
        zcat[:, 160:256] = zbuf[0:RS, 160:256]
        for qi in (1, 2, 3):
            zcat[:, FB * qi:FB * (qi + 1)] = zbuf[qi:qi + RS, :]
        acc = jnp.dot(zcat[...], wtap_ref[j],
                      preferred_element_type=jnp.float32)
        mmid = jnp.maximum(
            jnp.maximum(a[:, 0:FC - 2 * NCH], a[:, NCH:FC - NCH]),
            a[:, 2 * NCH:FC])
        mp0 = jnp.maximum(
            jnp.maximum(xact[XP - 1:XP - 1 + RS, FC - NCH:FC],
                        a[:, 0:NCH]), a[:, NCH:2 * NCH])
        mp7 = jnp.maximum(
            jnp.maximum(a[:, FC - 2 * NCH:FC - NCH], a[:, FC - NCH:FC]),
            xact[XP + 1:XP + 1 + RS, 0:NCH])
        mps = [mp0] + [mmid[:, NCH * (p - 1):NCH * p] for p in range(1, 7)] \
            + [mp7]
        mo = [jnp.dot(mps[p], wm_ref[j], preferred_element_type=jnp.float32)
              for p in range(F)]
        return acc + jnp.concatenate(mo, axis=1)

    store_act(z)
    a = big(0)
    store_act(jnp.maximum(a + vf_ref[6:7, :], 0.0))
    a = big(1)
    z = jnp.maximum(jnp.maximum(a + vf_ref[8:9, :], 0.0) + res1, 0.0)

    store_act(z)
    zb = z.astype(jnp.bfloat16)
    res2 = jnp.concatenate(
        [jnp.dot(zb[:, NCH * p:NCH * (p + 1)], r2_ref[...],
                 preferred_element_type=jnp.float32) for p in range(F)],
        axis=1)
    res2 += vf_ref[15:16, :]
    a = big(2)
    store_act(jnp.maximum(a + vf_ref[10:11, :], 0.0))
    a = big(3)
    store_act(jnp.maximum(a + vf_ref[12:13, :], 0.0))
    a = big(4)
    z = jnp.maximum(jnp.maximum(a + vf_ref[14:15, :], 0.0) + res2, 0.0)

    kr = lax.broadcasted_iota(jnp.int32, (RS, 1), 0)
    kr = kr - R * (kr // R)
    zm = jnp.where((kr >= RP) & (kr < RP + M), z, 0.0)
    pooled = []
    for b in range(BB):
        s1 = jnp.sum(zm[b * R:(b + 1) * R], axis=0, keepdims=True)
        p1 = s1[:, 0:NCH]
        for p in range(1, F):
            p1 = p1 + s1[:, NCH * p:NCH * (p + 1)]
        pooled.append(p1)
    pooled = jnp.concatenate(pooled, axis=0) * (1.0 / L)
    logits = jnp.dot(pooled, wf_ref[...], preferred_element_type=jnp.float32)
    logits += vf_ref[16:17, 0:NCH]
    col = lax.broadcasted_iota(jnp.int32, (BB, NCH), 1)
    logits = jnp.where(col < 2, logits, -1e30)
    mx = jnp.max(logits, axis=-1, keepdims=True)
    e = jnp.exp(logits - mx)
    out_ref[:, 0, :] = e * pl.reciprocal(jnp.sum(e, axis=-1, keepdims=True),
                                         approx=False)


@jax.jit
def _forward(x, w1c, wb_all, wc_all, wm_all, res2w, wf, vecs):
    B = x.shape[0]
    f32 = jnp.float32
    bf16 = jnp.bfloat16
    xf = jnp.pad(x.astype(f32).reshape(B, M, F), ((0, 0), (RP + 2, RP + 2),
                                                  (0, 0)))

    qs = jnp.arange(NQ)[:, None, None]
    ss = jnp.arange(F)[None, :, None]
    ps = jnp.arange(F)[None, None, :]
    t = 8 * (qs - 2) + ss - ps + 11
    valid = (t >= 0) & (t < NTAP)
    tc = jnp.clip(t, 0, NTAP - 1)
    eye8 = jnp.eye(F, dtype=f32)
    scale = vecs[jnp.array([5, 7, 9, 11, 13])]

    wc5 = wc_all.reshape(5, NTAP, NB, NCH)
    wtap = wc5[:, tc]
    wtap = jnp.where(valid[None, :, :, :, None, None], wtap, 0.0)
    wtap = wtap.transpose(0, 1, 2, 4, 3, 5)
    wtap = wtap * scale[:, None, None, None, None, :]
    wtap = wtap.reshape(5, NQ, FB, FC)
    slot0 = jnp.concatenate([wtap[:, 4, 0:96], jnp.zeros((5, 64, FC), f32),
                             wtap[:, 0, 160:256]], axis=1)
    wtapk = jnp.concatenate([slot0, wtap[:, 1], wtap[:, 2], wtap[:, 3]],
                            axis=1).astype(bf16)
    wbr = wb_all.astype(bf16)
    wmr = (wm_all * scale[:, None, :]).astype(bf16)
    r2r = res2w.astype(bf16)

    w1s = w1c * vecs[0][None, :]
    w1tap = jnp.where(valid[..., None], w1s[tc], 0.0)
    w1mp = jnp.einsum('ps,c->psc', eye8, vecs[2] * vecs[0])
    w1 = jnp.concatenate([w1tap.reshape(NQ * F, FC),
                          w1mp.reshape(F, FC),
                          jnp.zeros((NCH - NQ * F - F, FC), f32)], axis=0)
    wr1 = jnp.einsum('ps,c->psc', eye8, vecs[3]).reshape(F, FC)
    vf = jnp.tile(vecs, (1, F))

    flops_per_sample = (
        5 * (2 * L * NCH * NB + 2 * L * KTAP * NCH + 2 * L * NCH * NCH)
        + 2 * L * NCH + 2 * L * NCH * NCH + 2 * NCH * NCH)
    weight_bytes = (wtapk.size + wbr.size + wmr.size + r2r.size) * 2 \
        + (w1.size + wr1.size + wf.size + vf.size) * 4
    cost = pl.CostEstimate(flops=B * flops_per_sample,
                           transcendentals=B * NCH,
                           bytes_accessed=weight_bytes + int(xf.size) * 4
                           + B * NCH * 4)

    def resident(a):
        n = a.ndim
        return pl.BlockSpec(a.shape, lambda i: (0,) * n)

    out = pl.pallas_call(
        _kernel,
        out_shape=jax.ShapeDtypeStruct((B, 1, NCH), f32),
        grid=(B // BB,),
        in_specs=[pl.BlockSpec((BB, R + 4, F), lambda i: (i, 0, 0)),
                  resident(w1), resident(wr1), resident(wbr),
                  resident(wtapk), resident(wmr), resident(r2r),
                  resident(wf), resident(vf)],
        out_specs=pl.BlockSpec((BB, 1, NCH), lambda i: (i, 0, 0)),
        scratch_shapes=[pltpu.VMEM((RS + 2 * XP, FC), bf16),
                        pltpu.VMEM((RS + 4, FB), bf16),
                        pltpu.VMEM((RS, KTAP), bf16),
                        pltpu.VMEM((RS, NCH), f32)],
        compiler_params=pltpu.CompilerParams(
            dimension_semantics=("parallel",),
            vmem_limit_bytes=100 << 20),
        cost_estimate=cost,
    )(xf, w1, wr1, wbr, wtapk, wmr, r2r, wf, vf)
    return out[:, 0, :2]


def kernel(x, w1c, wb_all, wc_all, wm_all, res2w, wf, vecs):
    return _forward(x, w1c, wb_all, wc_all, wm_all, res2w, wf, vecs)


# speedup vs baseline: 1.2313x; 1.2313x over previous
import jax
import jax.numpy as jnp
from jax import lax
from jax.experimental import pallas as pl
from jax.experimental.pallas import tpu as pltpu

L = 1600
NCH = 128
NB = 32
F = 8
M = L // F
RP = 4
R = M + 2 * RP
FC = F * NCH
FB = F * NB
NQ = 5
KTAP = 4 * FB
NTAP = 23
BB = 4
RS = BB * R
XP = 8


def _kernel(x_ref, w1_ref, wr1_ref, wb_ref, wtap_ref, wm_ref, r2_ref, wf_ref,
            vf_ref, out_ref, xact, zbuf, zcat, xcat1):
    zbuf[0:2, :] = jnp.zeros((2, FB), jnp.bfloat16)
    zbuf[RS + 2:RS + 4, :] = jnp.zeros((2, FB), jnp.bfloat16)
    zcat[:, 96:160] = jnp.zeros((RS, 64), jnp.bfloat16)
    xact[0:XP, :] = jnp.zeros((XP, FC), jnp.bfloat16)
    xact[XP + RS:RS + 2 * XP, :] = jnp.zeros((XP, FC), jnp.bfloat16)
    xcat1[:, NQ * F + F:NCH] = jnp.zeros((RS, NCH - NQ * F - F), jnp.float32)

    for b in range(BB):
        for qi in range(NQ):
            xcat1[b * R:(b + 1) * R, F * qi:F * (qi + 1)] = \
                x_ref[b, qi:qi + R, :]
    xv = xcat1[:, 16:24]
    lf = jnp.concatenate([xcat1[:, 15:16], xcat1[:, 16:23]], axis=1)
    rf = jnp.concatenate([xcat1[:, 17:24], xcat1[:, 24:25]], axis=1)
    ki = lax.broadcasted_iota(jnp.int32, (RS, F), 0)
    ki = ki - R * (ki // R)
    ti = 8 * (ki - RP) + lax.broadcasted_iota(jnp.int32, (RS, F), 1)
    lf = jnp.where(ti == 0, -1e30, lf)
    rf = jnp.where(ti == L - 1, -1e30, rf)
    xcat1[:, NQ * F:NQ * F + F] = jnp.maximum(jnp.maximum(lf, xv), rf)
    acc = jnp.dot(xcat1[...], w1_ref[...], preferred_element_type=jnp.float32)
    z = jnp.maximum(acc + vf_ref[1:2, :], 0.0)
    res1 = jnp.dot(xv, wr1_ref[...], preferred_element_type=jnp.float32)
    res1 += vf_ref[4:5, :]

    def store_act(zval):
        xact[XP:XP + RS, :] = zval.astype(jnp.bfloat16)
        for b in range(BB):
            xact[XP + b * R:XP + b * R + RP, :] = \
                jnp.zeros((RP, FC), jnp.bfloat16)
            xact[XP + b * R + RP + M:XP + (b + 1) * R, :] = \
                jnp.zeros((RP, FC), jnp.bfloat16)

    def big(j):
        a = xact[XP:XP + RS, :]
        for s in range(F):
            zp = jnp.dot(a[:, NCH * s:NCH * (s + 1)], wb_ref[j],
                         preferred_element_type=jnp.float32)
            zbuf[2:2 + RS, NB * s:NB * (s + 1)] = zp.astype(jnp.bfloat16)
        zcat[:, 0:96] = zbuf[4:4 + RS, 0:96]
        zcat[:, 160:256] = zbuf[0:RS, 160:256]
        for qi in (1, 2, 3):
            zcat[:, FB * qi:FB * (qi + 1)] = zbuf[qi:qi + RS, :]
        acc = jnp.dot(zcat[...], wtap_ref[j],
                      preferred_element_type=jnp.float32)
        mmid = jnp.maximum(
            jnp.maximum(a[:, 0:FC - 2 * NCH], a[:, NCH:FC - NCH]),
            a[:, 2 * NCH:FC])
        mp0 = jnp.maximum(
            jnp.maximum(xact[XP - 1:XP - 1 + RS, FC - NCH:FC],
                        a[:, 0:NCH]), a[:, NCH:2 * NCH])
        mp7 = jnp.maximum(
            jnp.maximum(a[:, FC - 2 * NCH:FC - NCH], a[:, FC - NCH:FC]),
            xact[XP + 1:XP + 1 + RS, 0:NCH])
        mps = [mp0] + [mmid[:, NCH * (p - 1):NCH * p] for p in range(1, 7)] \
            + [mp7]
        mo = [jnp.dot(mps[p], wm_ref[j], preferred_element_type=jnp.float32)
              for p in range(F)]
        return acc + jnp.concatenate(mo, axis=1)

    store_act(z)
    a = big(0)
    store_act(jnp.maximum(a + vf_ref[6:7, :], 0.0))
    a = big(1)
    z = jnp.maximum(jnp.maximum(a + vf_ref[8:9, :], 0.0) + res1, 0.0)

    store_act(z)
    zb = z.astype(jnp.bfloat16)
    res2 = jnp.concatenate(
        [jnp.dot(zb[:, NCH * p:NCH * (p + 1)], r2_ref[...],
                 preferred_element_type=jnp.float32) for p in range(F)],
        axis=1)
    res2 += vf_ref[15:16, :]
    a = big(2)
    store_act(jnp.maximum(a + vf_ref[10:11, :], 0.0))
    a = big(3)
    store_act(jnp.maximum(a + vf_ref[12:13, :], 0.0))
    a = big(4)
    z = jnp.maximum(jnp.maximum(a + vf_ref[14:15, :], 0.0) + res2, 0.0)

    kr = lax.broadcasted_iota(jnp.int32, (RS, 1), 0)
    kr = kr - R * (kr // R)
    zm = jnp.where((kr >= RP) & (kr < RP + M), z, 0.0)
    pooled = []
    for b in range(BB):
        s1 = jnp.sum(zm[b * R:(b + 1) * R], axis=0, keepdims=True)
        p1 = s1[:, 0:NCH]
        for p in range(1, F):
            p1 = p1 + s1[:, NCH * p:NCH * (p + 1)]
        pooled.append(p1)
    pooled = jnp.concatenate(pooled, axis=0) * (1.0 / L)
    logits = jnp.dot(pooled, wf_ref[...], preferred_element_type=jnp.float32)
    logits += vf_ref[16:17, 0:NCH]
    col = lax.broadcasted_iota(jnp.int32, (BB, NCH), 1)
    logits = jnp.where(col < 2, logits, -1e30)
    mx = jnp.max(logits, axis=-1, keepdims=True)
    e = jnp.exp(logits - mx)
    out_ref[:, 0, :] = e * pl.reciprocal(jnp.sum(e, axis=-1, keepdims=True),
                                         approx=False)


@jax.jit
def _forward(x, w1c, wb_all, wc_all, wm_all, res2w, wf, vecs):
    B = x.shape[0]
    f32 = jnp.float32
    bf16 = jnp.bfloat16
    xf = jnp.pad(x.astype(f32).reshape(B, M, F), ((0, 0), (RP + 2, RP + 2),
                                                  (0, 0)))

    qs = jnp.arange(NQ)[:, None, None]
    ss = jnp.arange(F)[None, :, None]
    ps = jnp.arange(F)[None, None, :]
    t = 8 * (qs - 2) + ss - ps + 11
    valid = (t >= 0) & (t < NTAP)
    tc = jnp.clip(t, 0, NTAP - 1)
    eye8 = jnp.eye(F, dtype=f32)
    scale = vecs[jnp.array([5, 7, 9, 11, 13])]

    wc5 = wc_all.reshape(5, NTAP, NB, NCH)
    wtap = wc5[:, tc]
    wtap = jnp.where(valid[None, :, :, :, None, None], wtap, 0.0)
    wtap = wtap.transpose(0, 1, 2, 4, 3, 5)
    wtap = wtap * scale[:, None, None, None, None, :]
    wtap = wtap.reshape(5, NQ, FB, FC)
    slot0 = jnp.concatenate([wtap[:, 4, 0:96], jnp.zeros((5, 64, FC), f32),
                             wtap[:, 0, 160:256]], axis=1)
    wtapk = jnp.concatenate([slot0, wtap[:, 1], wtap[:, 2], wtap[:, 3]],
                            axis=1).astype(bf16)
    wbr = wb_all.astype(bf16)
    wmr = (wm_all * scale[:, None, :]).astype(bf16)
    r2r = res2w.astype(bf16)

    w1s = w1c * vecs[0][None, :]
    w1tap = jnp.where(valid[..., None], w1s[tc], 0.0)
    w1mp = jnp.einsum('ps,c->psc', eye8, vecs[2] * vecs[0])
    w1 = jnp.concatenate([w1tap.reshape(NQ * F, FC),
                          w1mp.reshape(F, FC),
                          jnp.zeros((NCH - NQ * F - F, FC), f32)], axis=0)
    wr1 = jnp.einsum('ps,c->psc', eye8, vecs[3]).reshape(F, FC)
    vf = jnp.tile(vecs, (1, F))

    flops_per_sample = (
        5 * (2 * L * NCH * NB + 2 * L * KTAP * NCH + 2 * L * NCH * NCH)
        + 2 * L * NCH + 2 * L * NCH * NCH + 2 * NCH * NCH)
    weight_bytes = (wtapk.size + wbr.size + wmr.size + r2r.size) * 2 \
        + (w1.size + wr1.size + wf.size + vf.size) * 4
    cost = pl.CostEstimate(flops=B * flops_per_sample,
                           transcendentals=B * NCH,
                           bytes_accessed=weight_bytes + int(xf.size) * 4
                           + B * NCH * 4)

    def resident(a):
        n = a.ndim
        return pl.BlockSpec(a.shape, lambda i: (0,) * n)

    out = pl.pallas_call(
        _kernel,
        out_shape=jax.ShapeDtypeStruct((B, 1, NCH), f32),
        grid=(B // BB,),
        in_specs=[pl.BlockSpec((BB, R + 4, F), lambda i: (i, 0, 0)),
                  resident(w1), resident(wr1), resident(wbr),
                  resident(wtapk), resident(wmr), resident(r2r),
                  resident(wf), resident(vf)],
        out_specs=pl.BlockSpec((BB, 1, NCH), lambda i: (i, 0, 0)),
        scratch_shapes=[pltpu.VMEM((RS + 2 * XP, FC), bf16),
                        pltpu.VMEM((RS + 4, FB), bf16),
                        pltpu.VMEM((RS, KTAP), bf16),
                        pltpu.VMEM((RS, NCH), f32)],
        compiler_params=pltpu.CompilerParams(
            dimension_semantics=("parallel",),
            vmem_limit_bytes=100 << 20),
        cost_estimate=cost,
    )(xf, w1, wr1, wbr, wtapk, wmr, r2r, wf, vf)
    return out[:, 0, :2]


def kernel(x, w1c, wb_all, wc_all, wm_all, res2w, wf, vecs):
    return _forward(x, w1c, wb_all, wc_all, wm_all, res2w, wf, vecs)
